# baseline (speedup 1.0000x reference)
import jax
import jax.numpy as jnp
from jax.experimental import pallas as pl
from jax.experimental.pallas import tpu as pltpu

_C = 1024
_NBUF = 4


def _ring_kernel(idx_ref, table_ref, x_hbm, o_hbm, ibuf, obuf, in_sems, out_sems):
    n = x_hbm.shape[0]
    nchunks = n // _C
    row = table_ref[idx_ref[0], :]

    def in_copy(j):
        s = j % _NBUF
        return pltpu.make_async_copy(
            x_hbm.at[pl.ds(j * _C, _C)], ibuf.at[s], in_sems.at[s])

    def out_copy(j):
        s = j % _NBUF
        return pltpu.make_async_copy(
            obuf.at[s], o_hbm.at[pl.ds(j * _C, _C)], out_sems.at[s])

    for k in range(min(_NBUF, nchunks)):
        in_copy(k).start()
    for j in range(nchunks):
        s = j % _NBUF
        in_copy(j).wait()
        if j >= _NBUF:
            out_copy(j - _NBUF).wait()
        obuf[s] = ibuf[s] + row[None, :]
        out_copy(j).start()
        if j + _NBUF < nchunks:
            in_copy(j + _NBUF).start()
    for j in range(max(nchunks - _NBUF, 0), nchunks):
        out_copy(j).wait()


def kernel(embeds, embed_type, table):
    b, s, h = embeds.shape
    n = b * s
    x = embeds.reshape(n, h)
    idx = jnp.asarray(embed_type, dtype=jnp.int32).reshape(1)
    out = pl.pallas_call(
        _ring_kernel,
        grid_spec=pltpu.PrefetchScalarGridSpec(
            num_scalar_prefetch=1,
            grid=(1,),
            in_specs=[
                pl.BlockSpec(table.shape, lambda i, idx_ref: (0, 0)),
                pl.BlockSpec(memory_space=pl.MemorySpace.ANY),
            ],
            out_specs=pl.BlockSpec(memory_space=pl.MemorySpace.ANY),
            scratch_shapes=[
                pltpu.VMEM((_NBUF, _C, h), jnp.float32),
                pltpu.VMEM((_NBUF, _C, h), jnp.float32),
                pltpu.SemaphoreType.DMA((_NBUF,)),
                pltpu.SemaphoreType.DMA((_NBUF,)),
            ],
        ),
        out_shape=jax.ShapeDtypeStruct((n, h), embeds.dtype),
        compiler_params=pltpu.CompilerParams(
            vmem_limit_bytes=67108864,
        ),
    )(idx, table, x)
    return out.reshape(b, s, h)


# speedup vs baseline: 1.0065x; 1.0065x over previous
import jax
import jax.numpy as jnp
from jax.experimental import pallas as pl
from jax.experimental.pallas import tpu as pltpu

_C = 2048
_NBUF = 3


def _ring_kernel(idx_ref, table_ref, x_hbm, o_hbm, ibuf, obuf, in_sems, out_sems):
    n = x_hbm.shape[0]
    nchunks = n // _C
    row = table_ref[idx_ref[0], :]

    def in_copy(j):
        s = j % _NBUF
        return pltpu.make_async_copy(
            x_hbm.at[pl.ds(j * _C, _C)], ibuf.at[s], in_sems.at[s])

    def out_copy(j):
        s = j % _NBUF
        return pltpu.make_async_copy(
            obuf.at[s], o_hbm.at[pl.ds(j * _C, _C)], out_sems.at[s])

    for k in range(min(_NBUF, nchunks)):
        in_copy(k).start()
    for j in range(nchunks):
        s = j % _NBUF
        in_copy(j).wait()
        if j >= _NBUF:
            out_copy(j - _NBUF).wait()
        obuf[s] = ibuf[s] + row[None, :]
        out_copy(j).start()
        if j + _NBUF < nchunks:
            in_copy(j + _NBUF).start()
    for j in range(max(nchunks - _NBUF, 0), nchunks):
        out_copy(j).wait()


def kernel(embeds, embed_type, table):
    b, s, h = embeds.shape
    n = b * s
    x = embeds.reshape(n, h)
    idx = jnp.asarray(embed_type, dtype=jnp.int32).reshape(1)
    out = pl.pallas_call(
        _ring_kernel,
        grid_spec=pltpu.PrefetchScalarGridSpec(
            num_scalar_prefetch=1,
            grid=(1,),
            in_specs=[
                pl.BlockSpec(table.shape, lambda i, idx_ref: (0, 0)),
                pl.BlockSpec(memory_space=pl.MemorySpace.ANY),
            ],
            out_specs=pl.BlockSpec(memory_space=pl.MemorySpace.ANY),
            scratch_shapes=[
                pltpu.VMEM((_NBUF, _C, h), jnp.float32),
                pltpu.VMEM((_NBUF, _C, h), jnp.float32),
                pltpu.SemaphoreType.DMA((_NBUF,)),
                pltpu.SemaphoreType.DMA((_NBUF,)),
            ],
        ),
        out_shape=jax.ShapeDtypeStruct((n, h), embeds.dtype),
        compiler_params=pltpu.CompilerParams(
            vmem_limit_bytes=67108864,
        ),
    )(idx, table, x)
    return out.reshape(b, s, h)


# speedup vs baseline: 1.0469x; 1.0402x over previous
import jax
import jax.numpy as jnp
from jax.experimental import pallas as pl
from jax.experimental.pallas import tpu as pltpu

_BM = 3840


def _add_row_kernel(idx_ref, table_ref, x_ref, o_ref):
    row = table_ref[idx_ref[0], :]
    o_ref[...] = x_ref[...] + row[None, :]


def kernel(embeds, embed_type, table):
    b, s, h = embeds.shape
    n = b * s
    x = embeds.reshape(n, h)
    idx = jnp.asarray(embed_type, dtype=jnp.int32).reshape(1)
    out = pl.pallas_call(
        _add_row_kernel,
        grid_spec=pltpu.PrefetchScalarGridSpec(
            num_scalar_prefetch=1,
            grid=(pl.cdiv(n, _BM),),
            in_specs=[
                pl.BlockSpec(table.shape, lambda i, idx_ref: (0, 0)),
                pl.BlockSpec((_BM, h), lambda i, idx_ref: (i, 0)),
            ],
            out_specs=pl.BlockSpec((_BM, h), lambda i, idx_ref: (i, 0)),
        ),
        out_shape=jax.ShapeDtypeStruct((n, h), embeds.dtype),
        compiler_params=pltpu.CompilerParams(
            dimension_semantics=("parallel",),
            vmem_limit_bytes=67108864,
        ),
    )(idx, table, x)
    return out.reshape(b, s, h)
